# baseline (speedup 1.0000x reference)
import functools

import jax
import jax.numpy as jnp
from jax import lax
from jax.experimental import pallas as pl
from jax.experimental.pallas import tpu as pltpu
from jax.experimental.pallas import tpu_sc as plsc

B = 16384
DD = 13
F = 26
VOCAB = 100000
E = 32

VSTRIDE = 100352
VB = 14336
NVB = VSTRIDE // VB
PROJ_N = F * VSTRIDE

NC = 2
NS = 16
NW = NC * NS
RPW = B // NW
CH = 16
NCHUNK = RPW // CH
SCALARS_C = CH * F
IDXS = 104
ROWS_PER_CHUNK = SCALARS_C // IDXS
IDX_ROWS_W = NCHUNK * ROWS_PER_CHUNK
SCALARS_W = RPW * F


def _tc_proj_body(t_ref, w_ref, o_ref):
    f = pl.program_id(0)
    o_ref[...] = jnp.sum(t_ref[0, :, :] * w_ref[f, :][:, None], axis=0)


def _sc_body(idx_hbm, proj_hbm, out_hbm, idx_v, vals_v, res_v, sem):
    cid = lax.axis_index("c")
    sid = lax.axis_index("s")
    wid = sid * NC + cid

    pltpu.sync_copy(idx_hbm.at[pl.ds(wid * IDX_ROWS_W, IDX_ROWS_W)], idx_v)

    def fire(j, carry):
        dst = pl.multiple_of(j * IDXS, 8)
        pltpu.async_copy(proj_hbm.at[idx_v.at[j]],
                         vals_v.at[pl.ds(dst, IDXS)], sem)
        return carry

    lax.fori_loop(0, IDX_ROWS_W, fire, 0)

    def drain(j, carry):
        pltpu.make_async_copy(
            proj_hbm.at[idx_v.at[0]], vals_v.at[pl.ds(0, IDXS)], sem).wait()
        return carry

    lax.fori_loop(0, IDX_ROWS_W, drain, 0)

    def chunk_body(c, carry):
        base = pl.multiple_of(c * SCALARS_C, 16)
        acc = jnp.zeros((16,), jnp.float32)
        for f in range(F):
            acc = acc + vals_v[pl.ds(base + f * CH, CH)]
        res_v[pl.ds(c * CH, CH)] = acc
        return carry

    lax.fori_loop(0, NCHUNK, chunk_body, 0)
    pltpu.sync_copy(res_v, out_hbm.at[pl.ds(wid * RPW, RPW)])


_sc_gather_sum = functools.partial(
    pl.kernel,
    out_type=jax.ShapeDtypeStruct((B,), jnp.float32),
    mesh=plsc.VectorSubcoreMesh(
        core_axis_name="c", subcore_axis_name="s",
        num_cores=NC, num_subcores=NS),
    compiler_params=pltpu.CompilerParams(
        needs_layout_passes=False, use_tc_tiling_on_sc=False),
    scratch_types=[
        pltpu.VMEM((IDX_ROWS_W, IDXS), jnp.int32),
        pltpu.VMEM((SCALARS_W,), jnp.float32),
        pltpu.VMEM((RPW,), jnp.float32),
        pltpu.SemaphoreType.DMA,
    ],
)(_sc_body)


def _tc_dense_body(x_ref, w_ref, b_ref, o_ref):
    o_ref[...] = jnp.sum(x_ref[...] * w_ref[...], axis=1) + b_ref[...]


def kernel(dense, cats, tables, W, b):
    tt = jnp.transpose(tables, (0, 2, 1))
    w2 = W[DD:, 0].reshape(F, E)

    proj = pl.pallas_call(
        _tc_proj_body,
        grid=(F, NVB),
        in_specs=[
            pl.BlockSpec((1, E, VB), lambda f, c: (f, 0, c)),
            pl.BlockSpec((F, E), lambda f, c: (0, 0)),
        ],
        out_specs=pl.BlockSpec((VB,), lambda f, c: (f * NVB + c,)),
        out_shape=jax.ShapeDtypeStruct((PROJ_N,), jnp.float32),
    )(tt, w2)

    idx26 = cats.T + (jnp.arange(F, dtype=jnp.int32) * VSTRIDE)[:, None]
    idxp = idx26.reshape(F, B // CH, CH).transpose(1, 0, 2)
    idxp = idxp.reshape(B // CH * ROWS_PER_CHUNK, IDXS)

    cat_part = _sc_gather_sum(idxp, proj)
    dense_part = pl.pallas_call(
        _tc_dense_body,
        out_shape=jax.ShapeDtypeStruct((B,), jnp.float32),
    )(dense, W[:DD, 0], b)

    return (cat_part + dense_part).reshape(B, 1)


# speedup vs baseline: 1.0380x; 1.0380x over previous
import functools

import jax
import jax.numpy as jnp
from jax import lax
from jax.experimental import pallas as pl
from jax.experimental.pallas import tpu as pltpu
from jax.experimental.pallas import tpu_sc as plsc

B = 16384
DD = 13
F = 26
VOCAB = 100000
E = 32

VSTRIDE = 100352
VB = 14336
NVB = VSTRIDE // VB
PROJ_N = F * VSTRIDE

NC = 2
NS = 16
NW = NC * NS
RPW = B // NW
CH = 16
NCHUNK = RPW // CH
SCALARS_C = CH * F
IDXS = 104
ROWS_PER_CHUNK = SCALARS_C // IDXS
IDX_ROWS_W = NCHUNK * ROWS_PER_CHUNK
SCALARS_W = RPW * F


def _tc_proj_body(t_ref, w_ref, o_ref):
    f = pl.program_id(0)
    o_ref[...] = jax.lax.dot_general(
        w_ref[f, :][None, :], t_ref[0, :, :],
        (((1,), (0,)), ((), ())),
        preferred_element_type=jnp.float32)[0]


def _sc_body(idx_hbm, proj_hbm, out_hbm, idx_v, vals_v, res_v, sem):
    cid = lax.axis_index("c")
    sid = lax.axis_index("s")
    wid = sid * NC + cid

    pltpu.sync_copy(idx_hbm.at[pl.ds(wid * IDX_ROWS_W, IDX_ROWS_W)], idx_v)

    def fire(j, carry):
        dst = pl.multiple_of(j * IDXS, 8)
        pltpu.async_copy(proj_hbm.at[idx_v.at[j]],
                         vals_v.at[pl.ds(dst, IDXS)], sem)
        return carry

    lax.fori_loop(0, IDX_ROWS_W, fire, 0)

    def drain(j, carry):
        pltpu.make_async_copy(
            proj_hbm.at[idx_v.at[0]], vals_v.at[pl.ds(0, IDXS)], sem).wait()
        return carry

    lax.fori_loop(0, IDX_ROWS_W, drain, 0)

    def chunk_body(c, carry):
        base = pl.multiple_of(c * SCALARS_C, 16)
        acc = jnp.zeros((16,), jnp.float32)
        for f in range(F):
            acc = acc + vals_v[pl.ds(base + f * CH, CH)]
        res_v[pl.ds(c * CH, CH)] = acc
        return carry

    lax.fori_loop(0, NCHUNK, chunk_body, 0)
    pltpu.sync_copy(res_v, out_hbm.at[pl.ds(wid * RPW, RPW)])


_sc_gather_sum = functools.partial(
    pl.kernel,
    out_type=jax.ShapeDtypeStruct((B,), jnp.float32),
    mesh=plsc.VectorSubcoreMesh(
        core_axis_name="c", subcore_axis_name="s",
        num_cores=NC, num_subcores=NS),
    compiler_params=pltpu.CompilerParams(
        needs_layout_passes=False, use_tc_tiling_on_sc=False),
    scratch_types=[
        pltpu.VMEM((IDX_ROWS_W, IDXS), jnp.int32),
        pltpu.VMEM((SCALARS_W,), jnp.float32),
        pltpu.VMEM((RPW,), jnp.float32),
        pltpu.SemaphoreType.DMA,
    ],
)(_sc_body)


def _tc_dense_body(x_ref, w_ref, b_ref, o_ref):
    o_ref[...] = jnp.sum(x_ref[...] * w_ref[...], axis=1) + b_ref[...]


def kernel(dense, cats, tables, W, b):
    tt = jnp.transpose(tables, (0, 2, 1))
    w2 = W[DD:, 0].reshape(F, E)

    proj = pl.pallas_call(
        _tc_proj_body,
        grid=(F, NVB),
        in_specs=[
            pl.BlockSpec((1, E, VB), lambda f, c: (f, 0, c)),
            pl.BlockSpec((F, E), lambda f, c: (0, 0)),
        ],
        out_specs=pl.BlockSpec((VB,), lambda f, c: (f * NVB + c,)),
        out_shape=jax.ShapeDtypeStruct((PROJ_N,), jnp.float32),
    )(tt, w2)

    idx26 = cats.T + (jnp.arange(F, dtype=jnp.int32) * VSTRIDE)[:, None]
    idxp = idx26.reshape(F, B // CH, CH).transpose(1, 0, 2)
    idxp = idxp.reshape(B // CH * ROWS_PER_CHUNK, IDXS)

    cat_part = _sc_gather_sum(idxp, proj)
    dense_part = pl.pallas_call(
        _tc_dense_body,
        out_shape=jax.ShapeDtypeStruct((B,), jnp.float32),
    )(dense, W[:DD, 0], b)

    return (cat_part + dense_part).reshape(B, 1)


# speedup vs baseline: 1.4630x; 1.4095x over previous
import functools

import jax
import jax.numpy as jnp
from jax import lax
from jax.experimental import pallas as pl
from jax.experimental.pallas import tpu as pltpu
from jax.experimental.pallas import tpu_sc as plsc

B = 16384
DD = 13
F = 26
VOCAB = 100000
E = 32

VSTRIDE = 100352
VB = 50176
NVB = VSTRIDE // VB
PROJ_N = F * VSTRIDE

NC = 2
NS = 16
NW = NC * NS
RPW = B // NW
CH = 16
NCHUNK = RPW // CH
SCALARS_C = CH * F
IDXS = 104
ROWS_PER_CHUNK = SCALARS_C // IDXS
IDX_ROWS_W = NCHUNK * ROWS_PER_CHUNK
SCALARS_W = RPW * F


def _tc_proj_body(t_ref, w_ref, o_ref):
    f = pl.program_id(0)
    o_ref[...] = jax.lax.dot_general(
        w_ref[f, :][None, :], t_ref[0, :, :],
        (((1,), (0,)), ((), ())),
        preferred_element_type=jnp.float32)[0]


def _sc_body(idx_hbm, proj_hbm, out_hbm, idx_v, vals_v, res_v, sem):
    cid = lax.axis_index("c")
    sid = lax.axis_index("s")
    wid = sid * NC + cid

    pltpu.sync_copy(idx_hbm.at[pl.ds(wid * IDX_ROWS_W, IDX_ROWS_W)], idx_v)

    def fire(j, carry):
        dst = pl.multiple_of(j * IDXS, 8)
        pltpu.async_copy(proj_hbm.at[idx_v.at[j]],
                         vals_v.at[pl.ds(dst, IDXS)], sem)
        return carry

    lax.fori_loop(0, IDX_ROWS_W, fire, 0)

    def drain(j, carry):
        pltpu.make_async_copy(
            proj_hbm.at[idx_v.at[0]], vals_v.at[pl.ds(0, IDXS)], sem).wait()
        return carry

    lax.fori_loop(0, IDX_ROWS_W, drain, 0)

    def chunk_body(c, carry):
        base = pl.multiple_of(c * SCALARS_C, 16)
        acc = jnp.zeros((16,), jnp.float32)
        for f in range(F):
            acc = acc + vals_v[pl.ds(base + f * CH, CH)]
        res_v[pl.ds(c * CH, CH)] = acc
        return carry

    lax.fori_loop(0, NCHUNK, chunk_body, 0)
    pltpu.sync_copy(res_v, out_hbm.at[pl.ds(wid * RPW, RPW)])


_sc_gather_sum = functools.partial(
    pl.kernel,
    out_type=jax.ShapeDtypeStruct((B,), jnp.float32),
    mesh=plsc.VectorSubcoreMesh(
        core_axis_name="c", subcore_axis_name="s",
        num_cores=NC, num_subcores=NS),
    compiler_params=pltpu.CompilerParams(
        needs_layout_passes=False, use_tc_tiling_on_sc=False),
    scratch_types=[
        pltpu.VMEM((IDX_ROWS_W, IDXS), jnp.int32),
        pltpu.VMEM((SCALARS_W,), jnp.float32),
        pltpu.VMEM((RPW,), jnp.float32),
        pltpu.SemaphoreType.DMA,
    ],
)(_sc_body)


def _tc_dense_body(x_ref, w_ref, b_ref, o_ref):
    o_ref[...] = jnp.sum(x_ref[...] * w_ref[...], axis=1) + b_ref[...]


def kernel(dense, cats, tables, W, b):
    tt = jnp.transpose(tables, (0, 2, 1))
    w2 = W[DD:, 0].reshape(F, E)

    proj = pl.pallas_call(
        _tc_proj_body,
        grid=(F, NVB),
        in_specs=[
            pl.BlockSpec((1, E, VB), lambda f, c: (f, 0, c)),
            pl.BlockSpec((F, E), lambda f, c: (0, 0)),
        ],
        out_specs=pl.BlockSpec((VB,), lambda f, c: (f * NVB + c,)),
        out_shape=jax.ShapeDtypeStruct((PROJ_N,), jnp.float32),
    )(tt, w2)

    idx26 = cats.T + (jnp.arange(F, dtype=jnp.int32) * VSTRIDE)[:, None]
    idxp = idx26.reshape(F, B // CH, CH).transpose(1, 0, 2)
    idxp = idxp.reshape(B // CH * ROWS_PER_CHUNK, IDXS)

    cat_part = _sc_gather_sum(idxp, proj)
    dense_part = pl.pallas_call(
        _tc_dense_body,
        out_shape=jax.ShapeDtypeStruct((B,), jnp.float32),
    )(dense, W[:DD, 0], b)

    return (cat_part + dense_part).reshape(B, 1)
